# baseline (speedup 1.0000x reference)
import functools

import jax
import jax.numpy as jnp
from jax import lax
from jax.experimental import pallas as pl
from jax.experimental.pallas import tpu as pltpu
from jax.experimental.pallas import tpu_sc as plsc

B, C, H, W = 8, 96, 128, 128
PH, PW = 7, 7
N = 512
NBINS = N * PH * PW
TAPS = 16
COLS = PH * PW * TAPS
NW = 32
CH = 8
ROWS_IDX = NBINS * TAPS // 128
CHUNKS_PER_W = ROWS_IDX // NW
NB = 64


def _splat(vec, k):
    dnums = lax.GatherDimensionNumbers(
        offset_dims=(), collapsed_slice_dims=(0,), start_index_map=(0,))
    idx = jnp.full((16, 1), k, jnp.int32)
    return lax.gather(vec, idx, dnums, slice_sizes=(1,),
                      mode=lax.GatherScatterMode.PROMISE_IN_BOUNDS)


def _prep_body(boxes_ref, bids_ref, idx_ref, w_ref):
    f = jnp.float32
    col = lax.broadcasted_iota(jnp.int32, (NB, COLS), 1)
    colbin = col >> 4
    tap = col & 15
    iy = (tap >> 3) & 1
    ix = (tap >> 2) & 1
    dy = (tap >> 1) & 1
    dx = tap & 1
    py = jnp.floor(colbin.astype(f) * (1.0 / 7.0)).astype(jnp.int32)
    px = colbin - 7 * py

    x1 = boxes_ref[:, 0:1] - 0.5
    y1 = boxes_ref[:, 1:2] - 0.5
    x2 = boxes_ref[:, 2:3] - 0.5
    y2 = boxes_ref[:, 3:4] - 0.5
    roi_w = x2 - x1
    roi_h = y2 - y1
    bin_h = roi_h / PH
    bin_w = roi_w / PW
    grid_h = jnp.ceil(roi_h / PH).astype(jnp.int32)
    grid_w = jnp.ceil(roi_w / PW).astype(jnp.int32)
    count = jnp.maximum(grid_h * grid_w, 1).astype(f)
    gh = jnp.maximum(grid_h, 1).astype(f)
    gw = jnp.maximum(grid_w, 1).astype(f)

    ys = y1 + py.astype(f) * bin_h + (iy.astype(f) + 0.5) * bin_h / gh
    xs = x1 + px.astype(f) * bin_w + (ix.astype(f) + 0.5) * bin_w / gw
    mask = (iy < grid_h) & (ix < grid_w)
    inb = (ys >= -1.0) & (ys <= H) & (xs >= -1.0) & (xs <= W)
    y = jnp.maximum(ys, 0.0)
    x = jnp.maximum(xs, 0.0)
    y0 = jnp.floor(y).astype(jnp.int32)
    x0 = jnp.floor(x).astype(jnp.int32)
    y0c = jnp.minimum(y0, H - 1)
    x0c = jnp.minimum(x0, W - 1)
    y1i = jnp.minimum(y0c + 1, H - 1)
    x1i = jnp.minimum(x0c + 1, W - 1)
    y = jnp.where(y0 >= H - 1, y0c.astype(f), y)
    x = jnp.where(x0 >= W - 1, x0c.astype(f), x)
    ly = y - y0c.astype(f)
    lx = x - x0c.astype(f)
    wy = jnp.where(dy == 1, ly, 1.0 - ly)
    wx = jnp.where(dx == 1, lx, 1.0 - lx)
    valid = (mask & inb).astype(f)
    w_ref[:, :] = wy * wx * valid / count
    ty = jnp.where(dy == 1, y1i, y0c)
    tx = jnp.where(dx == 1, x1i, x0c)
    bid = bids_ref[:, 0:1].astype(jnp.int32)
    idx_ref[:, :] = bid * (H * W) + ty * W + tx


_precompute = pl.pallas_call(
    _prep_body,
    grid=(N // NB,),
    in_specs=[
        pl.BlockSpec((NB, 4), lambda i: (i, 0)),
        pl.BlockSpec((NB, 1), lambda i: (i, 0)),
    ],
    out_specs=[
        pl.BlockSpec((NB, COLS), lambda i: (i, 0)),
        pl.BlockSpec((NB, COLS), lambda i: (i, 0)),
    ],
    out_shape=[
        jax.ShapeDtypeStruct((N, COLS), jnp.int32),
        jax.ShapeDtypeStruct((N, COLS), jnp.float32),
    ],
)


@functools.partial(
    pl.kernel,
    mesh=plsc.VectorSubcoreMesh(core_axis_name="c", subcore_axis_name="s"),
    out_type=jax.ShapeDtypeStruct((NBINS, C), jnp.float32),
    scratch_types=[
        pltpu.VMEM((CHUNKS_PER_W, 128), jnp.int32),
        pltpu.VMEM((CHUNKS_PER_W, 128), jnp.float32),
        pltpu.VMEM((256, 128), jnp.float32),
        pltpu.VMEM((256, 128), jnp.float32),
        pltpu.VMEM((16, C), jnp.float32),
        pltpu.VMEM((16, C), jnp.float32),
        pltpu.SemaphoreType.DMA,
        pltpu.SemaphoreType.DMA,
        pltpu.SemaphoreType.DMA,
        pltpu.SemaphoreType.DMA,
    ],
)
def _sc_pool(ft_hbm, idx_hbm, w_hbm, out_hbm, idx_v, w_sv, rows0, rows1,
             out0, out1, sg0, sg1, ss0, ss1):
    wid = lax.axis_index("s") * 2 + lax.axis_index("c")
    nbc = CHUNKS_PER_W // 2
    base = wid * nbc
    rows = (rows0, rows1)
    outs = (out0, out1)
    sg = (sg0, sg1)
    ss = (ss0, ss1)

    pltpu.sync_copy(idx_hbm.at[wid], idx_v)
    pltpu.sync_copy(w_hbm.at[wid], w_sv)

    def gather_half(c, b, h):
        return pltpu.make_async_copy(
            ft_hbm.at[idx_v.at[2 * c + h]],
            rows[b].at[pl.ds(h * 128, 128)], sg[b])

    def gather_start(c, b):
        gather_half(c, b, 0).start()
        gather_half(c, b, 1).start()

    def gather_wait(c, b):
        gather_half(c, b, 0).wait()
        gather_half(c, b, 1).wait()

    def store(c, b):
        return pltpu.make_async_copy(
            outs[b], out_hbm.at[pl.ds((base + c) * 16, 16)], ss[b])

    def compute(c, b):
        rv = rows[b]
        ov = outs[b]
        for j in range(16):
            wvec = w_sv[2 * c + j // 8, pl.ds((j % 8) * 16, 16)]
            ws = [_splat(wvec, k) for k in range(TAPS)]
            for cb in range(C // 16):
                sl = pl.ds(cb * 16, 16)
                acc = ws[0] * rv[j * 16, sl]
                for k in range(1, TAPS):
                    acc = acc + ws[k] * rv[j * 16 + k, sl]
                ov[j, sl] = acc

    gather_start(0, 0)

    def body(i, carry):
        for u in range(2):
            c = 2 * i + u

            @pl.when(c + 1 < nbc)
            def _():
                gather_start(c + 1, 1 - u)

            gather_wait(c, u)

            @pl.when(c >= 2)
            def _():
                store(c - 2, u).wait()

            compute(c, u)
            store(c, u).start()
        return carry

    lax.fori_loop(0, nbc // 2, body, 0)
    c = nbc - 1
    gather_wait(c, 0)
    store(c - 2, 0).wait()
    compute(c, 0)
    store(c, 0).start()
    store(c - 1, 1).wait()
    store(c, 0).wait()


def kernel(features, boxes, batch_ids):
    ft = jnp.transpose(features, (0, 2, 3, 1)).reshape(B * H * W, C)
    ft = jnp.pad(ft, ((0, 0), (0, 128 - C)))
    idx, wt = _precompute(boxes, batch_ids.reshape(N, 1))
    out = _sc_pool(ft, idx.reshape(NW, CHUNKS_PER_W, 128),
                   wt.reshape(NW, CHUNKS_PER_W, 128))
    return jnp.transpose(out.reshape(N, PH, PW, C), (0, 3, 1, 2))


# speedup vs baseline: 1.0786x; 1.0786x over previous
import functools

import jax
import jax.numpy as jnp
from jax import lax
from jax.experimental import pallas as pl
from jax.experimental.pallas import tpu as pltpu
from jax.experimental.pallas import tpu_sc as plsc

B, C, H, W = 8, 96, 128, 128
PH, PW = 7, 7
N = 512
NBINS = N * PH * PW
TAPS = 16
COLS = PH * PW * TAPS
NW = 32
CH = 8
ROWS_IDX = NBINS * TAPS // 128
CHUNKS_PER_W = ROWS_IDX // NW
NB = 64


def _splat(vec, k):
    dnums = lax.GatherDimensionNumbers(
        offset_dims=(), collapsed_slice_dims=(0,), start_index_map=(0,))
    idx = jnp.full((16, 1), k, jnp.int32)
    return lax.gather(vec, idx, dnums, slice_sizes=(1,),
                      mode=lax.GatherScatterMode.PROMISE_IN_BOUNDS)


def _prep_body(boxes_ref, bids_ref, idx_ref, w_ref):
    f = jnp.float32
    col = lax.broadcasted_iota(jnp.int32, (NB, COLS), 1)
    colbin = col >> 4
    tap = col & 15
    iy = (tap >> 3) & 1
    ix = (tap >> 2) & 1
    dy = (tap >> 1) & 1
    dx = tap & 1
    py = jnp.floor(colbin.astype(f) * (1.0 / 7.0)).astype(jnp.int32)
    px = colbin - 7 * py

    x1 = boxes_ref[:, 0:1] - 0.5
    y1 = boxes_ref[:, 1:2] - 0.5
    x2 = boxes_ref[:, 2:3] - 0.5
    y2 = boxes_ref[:, 3:4] - 0.5
    roi_w = x2 - x1
    roi_h = y2 - y1
    bin_h = roi_h / PH
    bin_w = roi_w / PW
    grid_h = jnp.ceil(roi_h / PH).astype(jnp.int32)
    grid_w = jnp.ceil(roi_w / PW).astype(jnp.int32)
    count = jnp.maximum(grid_h * grid_w, 1).astype(f)
    gh = jnp.maximum(grid_h, 1).astype(f)
    gw = jnp.maximum(grid_w, 1).astype(f)

    ys = y1 + py.astype(f) * bin_h + (iy.astype(f) + 0.5) * bin_h / gh
    xs = x1 + px.astype(f) * bin_w + (ix.astype(f) + 0.5) * bin_w / gw
    mask = (iy < grid_h) & (ix < grid_w)
    inb = (ys >= -1.0) & (ys <= H) & (xs >= -1.0) & (xs <= W)
    y = jnp.maximum(ys, 0.0)
    x = jnp.maximum(xs, 0.0)
    y0 = jnp.floor(y).astype(jnp.int32)
    x0 = jnp.floor(x).astype(jnp.int32)
    y0c = jnp.minimum(y0, H - 1)
    x0c = jnp.minimum(x0, W - 1)
    y1i = jnp.minimum(y0c + 1, H - 1)
    x1i = jnp.minimum(x0c + 1, W - 1)
    y = jnp.where(y0 >= H - 1, y0c.astype(f), y)
    x = jnp.where(x0 >= W - 1, x0c.astype(f), x)
    ly = y - y0c.astype(f)
    lx = x - x0c.astype(f)
    wy = jnp.where(dy == 1, ly, 1.0 - ly)
    wx = jnp.where(dx == 1, lx, 1.0 - lx)
    valid = (mask & inb).astype(f)
    w_ref[:, :] = wy * wx * valid / count
    ty = jnp.where(dy == 1, y1i, y0c)
    tx = jnp.where(dx == 1, x1i, x0c)
    bid = bids_ref[:, 0:1].astype(jnp.int32)
    idx_ref[:, :] = bid * (H * W) + ty * W + tx


_precompute = pl.pallas_call(
    _prep_body,
    grid=(N // NB,),
    in_specs=[
        pl.BlockSpec((NB, 4), lambda i: (i, 0)),
        pl.BlockSpec((NB, 1), lambda i: (i, 0)),
    ],
    out_specs=[
        pl.BlockSpec((NB, COLS), lambda i: (i, 0)),
        pl.BlockSpec((NB, COLS), lambda i: (i, 0)),
    ],
    out_shape=[
        jax.ShapeDtypeStruct((N, COLS), jnp.int32),
        jax.ShapeDtypeStruct((N, COLS), jnp.float32),
    ],
)


@functools.partial(
    pl.kernel,
    mesh=plsc.VectorSubcoreMesh(core_axis_name="c", subcore_axis_name="s"),
    out_type=jax.ShapeDtypeStruct((NBINS, C), jnp.float32),
    scratch_types=[
        pltpu.VMEM((CHUNKS_PER_W, 128), jnp.int32),
        pltpu.VMEM((CHUNKS_PER_W, 128), jnp.float32),
        pltpu.VMEM((128, 128), jnp.float32),
        pltpu.VMEM((128, 128), jnp.float32),
        pltpu.VMEM((128, 128), jnp.float32),
        pltpu.VMEM((128, 128), jnp.float32),
        pltpu.VMEM((CH, C), jnp.float32),
        pltpu.VMEM((CH, C), jnp.float32),
        pltpu.SemaphoreType.DMA,
        pltpu.SemaphoreType.DMA,
        pltpu.SemaphoreType.DMA,
        pltpu.SemaphoreType.DMA,
        pltpu.SemaphoreType.DMA,
        pltpu.SemaphoreType.DMA,
    ],
)
def _sc_pool(ft_hbm, idx_hbm, w_hbm, out_hbm, idx_v, w_sv, rows0, rows1,
             rows2, rows3, out0, out1, sg0, sg1, sg2, sg3, ss0, ss1):
    wid = lax.axis_index("s") * 2 + lax.axis_index("c")
    base = wid * CHUNKS_PER_W
    rows = (rows0, rows1, rows2, rows3)
    outs = (out0, out1)
    sg = (sg0, sg1, sg2, sg3)
    ss = (ss0, ss1)

    pltpu.sync_copy(idx_hbm.at[wid], idx_v)
    pltpu.sync_copy(w_hbm.at[wid], w_sv)

    def gather(c, b):
        return pltpu.make_async_copy(ft_hbm.at[idx_v.at[c]], rows[b], sg[b])

    def store(c, b):
        return pltpu.make_async_copy(
            outs[b], out_hbm.at[pl.ds((base + c) * CH, CH)], ss[b])

    def compute(c, b, ob):
        rv = rows[b]
        ov = outs[ob]
        for j in range(CH):
            wvec = w_sv[c, pl.ds(j * 16, 16)]
            ws = [_splat(wvec, k) for k in range(TAPS)]
            for cb in range(C // 16):
                sl = pl.ds(cb * 16, 16)
                acc = ws[0] * rv[j * 16, sl]
                for k in range(1, TAPS):
                    acc = acc + ws[k] * rv[j * 16 + k, sl]
                ov[j, sl] = acc

    gather(0, 0).start()
    gather(1, 1).start()
    gather(2, 2).start()

    def body(i, carry):
        for u in range(4):
            c = 4 * i + u
            pre = c + 3

            @pl.when(pre < CHUNKS_PER_W)
            def _():
                gather(pre, (u + 3) % 4).start()

            gather(c, u).wait()

            @pl.when(c >= 2)
            def _():
                store(c - 2, u % 2).wait()

            compute(c, u, u % 2)
            store(c, u % 2).start()
        return carry

    lax.fori_loop(0, CHUNKS_PER_W // 4, body, 0)
    for c, u in ((CHUNKS_PER_W - 2, 0), (CHUNKS_PER_W - 1, 1)):
        gather(c, u).wait()
        store(c - 2, u % 2).wait()
        compute(c, u, u % 2)
        store(c, u % 2).start()
    store(CHUNKS_PER_W - 2, 0).wait()
    store(CHUNKS_PER_W - 1, 1).wait()


def kernel(features, boxes, batch_ids):
    ft = jnp.transpose(features, (0, 2, 3, 1)).reshape(B * H * W, C)
    ft = jnp.pad(ft, ((0, 0), (0, 128 - C)))
    idx, wt = _precompute(boxes, batch_ids.reshape(N, 1))
    out = _sc_pool(ft, idx.reshape(NW, CHUNKS_PER_W, 128),
                   wt.reshape(NW, CHUNKS_PER_W, 128))
    return jnp.transpose(out.reshape(N, PH, PW, C), (0, 3, 1, 2))
